# baseline (speedup 1.0000x reference)
import functools

import jax
import jax.numpy as jnp
from jax import lax
from jax.experimental import pallas as pl
from jax.experimental.pallas import tpu as pltpu
from jax.experimental.pallas import tpu_sc as plsc

D_MODEL = 64
BATCH = 1024
LENGTH = 200
NUM_WORKERS = 32
ROWS_PER_W = BATCH // NUM_WORKERS


@functools.partial(
    pl.kernel,
    out_type=jax.ShapeDtypeStruct((BATCH, LENGTH, D_MODEL), jnp.float32),
    mesh=plsc.VectorSubcoreMesh(core_axis_name="c", subcore_axis_name="s"),
    compiler_params=pltpu.CompilerParams(use_tc_tiling_on_sc=False),
    scratch_types=[
        pltpu.VMEM((ROWS_PER_W, LENGTH), jnp.int32),
        pltpu.VMEM((LENGTH, D_MODEL), jnp.float32),
        pltpu.VMEM((LENGTH, D_MODEL), jnp.float32),
        pltpu.SemaphoreType.DMA,
        pltpu.SemaphoreType.DMA,
        pltpu.SemaphoreType.DMA,
        pltpu.SemaphoreType.DMA,
    ],
)
def _embed_gather(ids_hbm, table_hbm, out_hbm, idx_v, rows0, rows1,
                  gsem0, gsem1, osem0, osem1):
    wid = lax.axis_index("s") * 2 + lax.axis_index("c")
    base = wid * ROWS_PER_W
    pltpu.sync_copy(ids_hbm.at[pl.ds(base, ROWS_PER_W)], idx_v)

    bufs = (rows0, rows1)
    gsems = (gsem0, gsem1)
    osems = (osem0, osem1)

    def start_gather(r):
        return pltpu.async_copy(
            table_hbm.at[idx_v.at[r]], bufs[r % 2], gsems[r % 2])

    def start_write(r):
        return pltpu.async_copy(
            bufs[r % 2], out_hbm.at[base + r], osems[r % 2])

    gathers = [start_gather(0), start_gather(1)]
    writes = [None, None]
    for r in range(ROWS_PER_W):
        gathers[r % 2].wait()
        writes[r % 2] = start_write(r)
        nxt = r + 2
        if nxt < ROWS_PER_W:
            writes[nxt % 2].wait()
            gathers[nxt % 2] = start_gather(nxt)
    writes[(ROWS_PER_W - 2) % 2].wait()
    writes[(ROWS_PER_W - 1) % 2].wait()


def kernel(ids, emb_weight):
    return _embed_gather(ids, emb_weight)
